# baseline (speedup 1.0000x reference)
import functools

import jax
import jax.numpy as jnp
from jax import lax
from jax.experimental import pallas as pl
from jax.experimental.pallas import tpu as pltpu
from jax.experimental.pallas import tpu_sc as plsc

VOCAB = 1000000
EMBED_DIM = 16
BATCH = 16384
FIELDS = 26
N = BATCH * FIELDS
NUM_CORES = 2
NUM_SUBCORES = 16
NW = NUM_CORES * NUM_SUBCORES
GW = 128
G = N // GW
G_PER_W = G // NW
K = 13
CHUNK = K * GW
NCHUNK = G_PER_W // K
L = 16

RC = 2048
NRC = (VOCAB + RC - 1) // RC
RC_LAST = VOCAB - (NRC - 1) * RC
RC_PER_W = (NRC + NW - 1) // NW

_SC_PARAMS = pltpu.CompilerParams(
    use_tc_tiling_on_sc=False, needs_layout_passes=False)


def _make_relayout():
    mesh = plsc.VectorSubcoreMesh(core_axis_name="c", subcore_axis_name="s")

    @functools.partial(
        pl.kernel,
        mesh=mesh,
        out_type=jax.ShapeDtypeStruct((VOCAB, EMBED_DIM), jnp.float32),
        scratch_types=[
            pltpu.VMEM((EMBED_DIM, RC), jnp.float32),
            pltpu.VMEM((RC, EMBED_DIM), jnp.float32),
            pltpu.SemaphoreType.DMA,
        ],
        compiler_params=_SC_PARAMS,
    )
    def k(planes_hbm, rows_hbm, pin_v, rout_v, sem):
        wid = lax.axis_index("s") * NUM_CORES + lax.axis_index("c")
        lane = lax.iota(jnp.int32, L)
        RUN = 16

        def do_chunk(cid, nrows):
            off = cid * RC
            pltpu.sync_copy(planes_hbm.at[:, pl.ds(off, nrows)],
                            pin_v.at[:, pl.ds(0, nrows)])

            def group(g, carry2):
                i0 = g * RUN
                for r in range(RUN):
                    rout_v[i0 + r, :] = plsc.load_gather(
                        pin_v, [lane, jnp.full((L,), i0 + r, jnp.int32)])
                return carry2

            lax.fori_loop(0, nrows // RUN, group, 0)
            pltpu.sync_copy(rout_v.at[pl.ds(0, nrows)],
                            rows_hbm.at[pl.ds(off, nrows)])

        def chunk(c, carry):
            cid = wid + c * NW

            @pl.when(cid < NRC - 1)
            def _():
                do_chunk(cid, RC)

            @pl.when(cid == NRC - 1)
            def _():
                do_chunk(cid, RC_LAST)

            return carry

        lax.fori_loop(0, RC_PER_W, chunk, 0)

    return k


def _make_gather():
    mesh = plsc.VectorSubcoreMesh(core_axis_name="c", subcore_axis_name="s")

    @functools.partial(
        pl.kernel,
        mesh=mesh,
        out_type=jax.ShapeDtypeStruct((EMBED_DIM, N), jnp.float32),
        scratch_types=[
            pltpu.VMEM((G_PER_W, GW), jnp.int32),
            pltpu.VMEM((2, CHUNK, EMBED_DIM), jnp.float32),
            pltpu.VMEM((2, EMBED_DIM, CHUNK), jnp.float32),
            pltpu.SemaphoreType.DMA,
            pltpu.SemaphoreType.DMA,
        ],
        compiler_params=_SC_PARAMS,
    )
    def k(idx_hbm, table_hbm, out_hbm, idx_v, rows_v, planes_v, gsem, wsem):
        wid = lax.axis_index("s") * NUM_CORES + lax.axis_index("c")
        gbase = wid * G_PER_W
        fbase = gbase * GW
        pltpu.sync_copy(idx_hbm.at[pl.ds(gbase, G_PER_W)], idx_v)

        def fire(c, buf):
            for j in range(K):
                pltpu.async_copy(
                    table_hbm.at[idx_v.at[c * K + j]],
                    rows_v.at[buf, pl.ds(j * GW, GW), :], gsem)

        def drain_gathers(buf):
            for j in range(K):
                pltpu.make_async_copy(
                    table_hbm.at[idx_v.at[j]],
                    rows_v.at[buf, pl.ds(j * GW, GW), :], gsem).wait()

        def plane_start(c, buf):
            pltpu.async_copy(
                planes_v.at[buf],
                out_hbm.at[:, pl.ds(fbase + c * CHUNK, CHUNK)], wsem)

        def plane_wait(c, buf):
            pltpu.make_async_copy(
                planes_v.at[buf],
                out_hbm.at[:, pl.ds(fbase + c * CHUNK, CHUNK)], wsem).wait()

        fire(0, 0)
        lane = lax.iota(jnp.int32, L)
        RUN = 16

        def chunk(c, carry):
            buf = lax.rem(c, 2)

            @pl.when(c + 1 < NCHUNK)
            def _():
                fire(c + 1, 1 - buf)

            drain_gathers(buf)

            @pl.when(c >= 2)
            def _():
                plane_wait(c, buf)

            rows = rows_v.at[buf]
            planes = planes_v.at[buf]

            def group(g, carry2):
                i0 = g * RUN
                for r in range(RUN):
                    v = rows[i0 + r, :]
                    plsc.store_scatter(
                        planes, [lane, jnp.full((L,), i0 + r, jnp.int32)], v)
                return carry2

            lax.fori_loop(0, CHUNK // RUN, group, 0)
            plane_start(c, buf)
            return carry

        lax.fori_loop(0, NCHUNK, chunk, 0)
        plane_wait(NCHUNK - 2, 0)
        plane_wait(NCHUNK - 1, 1)

    return k


_relayout = _make_relayout()
_embed_gather = _make_gather()


def kernel(x, table):
    idx = x.T.reshape(G, GW).astype(jnp.int32)
    rowtab = _relayout(table.T)
    planes = _embed_gather(idx, rowtab)
    return planes.reshape(EMBED_DIM, FIELDS, BATCH).transpose(2, 1, 0)


# speedup vs baseline: 2.9767x; 2.9767x over previous
import functools

import jax
import jax.numpy as jnp
from jax import lax
from jax.experimental import pallas as pl
from jax.experimental.pallas import tpu as pltpu
from jax.experimental.pallas import tpu_sc as plsc

VOCAB = 1000000
EMBED_DIM = 16
BATCH = 16384
FIELDS = 26
N = BATCH * FIELDS
NUM_CORES = 2
NUM_SUBCORES = 16
NW = NUM_CORES * NUM_SUBCORES
GW = 128
G = N // GW
G_PER_W = G // NW
K = 2
CH = K * GW
NCHUNK = G_PER_W // K
L = 16
TROWS = VOCAB // 8


def _make_kernel():
    mesh = plsc.VectorSubcoreMesh(core_axis_name="c", subcore_axis_name="s")

    @functools.partial(
        pl.kernel,
        mesh=mesh,
        out_type=jax.ShapeDtypeStruct((EMBED_DIM, N), jnp.float32),
        scratch_types=[
            pltpu.VMEM((G_PER_W, GW), jnp.int32),
            pltpu.VMEM((2, K, GW), jnp.int32),
            pltpu.VMEM((2, CH, GW), jnp.float32),
            pltpu.VMEM((2, EMBED_DIM, CH), jnp.float32),
            pltpu.SemaphoreType.DMA,
            pltpu.SemaphoreType.DMA,
        ],
        compiler_params=pltpu.CompilerParams(
            use_tc_tiling_on_sc=True, needs_layout_passes=False),
    )
    def k(idx_hbm, tab_hbm, out_hbm, idx_v, gidx_v, grp_v, planes_v,
          gsem, wsem):
        wid = lax.axis_index("s") * NUM_CORES + lax.axis_index("c")
        gbase = wid * G_PER_W
        fbase = gbase * GW
        lane = lax.iota(jnp.int32, L)
        pltpu.sync_copy(idx_hbm.at[pl.ds(gbase, G_PER_W)], idx_v)

        def make_gidx(c, buf):
            for s in range(K):
                row = c * K + s
                for q in range(GW // L):
                    col = lane + q * L
                    v = plsc.load_gather(idx_v, [jnp.full((L,), row,
                                                          jnp.int32), col])
                    plsc.store_scatter(
                        gidx_v, [jnp.full((L,), buf, jnp.int32),
                                 jnp.full((L,), s, jnp.int32), col],
                        lax.shift_right_logical(v, 3))

        def fire(buf):
            for s in range(K):
                pltpu.async_copy(
                    tab_hbm.at[gidx_v.at[buf, s]],
                    grp_v.at[buf, pl.ds(s * GW, GW), :], gsem)

        def drain(buf):
            for s in range(K):
                pltpu.make_async_copy(
                    tab_hbm.at[gidx_v.at[buf, s]],
                    grp_v.at[buf, pl.ds(s * GW, GW), :], gsem).wait()

        def plane_start(c, buf):
            pltpu.async_copy(
                planes_v.at[buf],
                out_hbm.at[:, pl.ds(fbase + c * CH, CH)], wsem)

        def plane_wait(c, buf):
            pltpu.make_async_copy(
                planes_v.at[buf],
                out_hbm.at[:, pl.ds(fbase + c * CH, CH)], wsem).wait()

        make_gidx(0, 0)
        fire(0)

        def chunk(c, carry):
            buf = lax.rem(c, 2)

            @pl.when(c + 1 < NCHUNK)
            def _():
                make_gidx(c + 1, 1 - buf)
                fire(1 - buf)

            drain(buf)

            @pl.when(c >= 2)
            def _():
                plane_wait(c, buf)

            grp = grp_v.at[buf]
            planes = planes_v.at[buf]

            def step(t, carry2):
                s0 = t * L
                row = c * K + t // 8
                col = lane + lax.rem(t, 8) * L
                idxv = plsc.load_gather(
                    idx_v, [jnp.full((L,), row, jnp.int32), col])
                sub = lax.rem(idxv, 8) * EMBED_DIM
                slot = lane + s0
                for j in range(EMBED_DIM):
                    vj = plsc.load_gather(grp, [slot, sub + j])
                    plsc.store_scatter(
                        planes, [jnp.full((L,), j, jnp.int32), slot], vj)
                return carry2

            lax.fori_loop(0, CH // L, step, 0)
            plane_start(c, buf)
            return carry

        lax.fori_loop(0, NCHUNK, chunk, 0)
        plane_wait(NCHUNK - 2, 0)
        plane_wait(NCHUNK - 1, 1)

    return k


_embed_gather = _make_kernel()


def kernel(x, table):
    idx = x.T.reshape(G, GW).astype(jnp.int32)
    tab = table.reshape(TROWS, 8 * EMBED_DIM)
    planes = _embed_gather(idx, tab)
    return planes.reshape(EMBED_DIM, FIELDS, BATCH).transpose(2, 1, 0)
